# Initial kernel scaffold; baseline (speedup 1.0000x reference)
import jax
import jax.numpy as jnp
from jax.experimental import pallas as pl


def kernel(poi_rep, edge_index, dist_vec, W, b):
    raise NotImplementedError("write your pallas kernel here")


# speedup vs baseline: 7.2247x; 7.2247x over previous
import dataclasses
import functools

import jax
import jax.numpy as jnp
from jax import lax
from jax.experimental import pallas as pl
from jax.experimental.pallas import tpu as pltpu
from jax.experimental.pallas import tpu_sc as plsc

N = 10000
E = 320000
D = 128

NC = 2
NS = 16
L = 16
NW = NC * NS
E_TILE = E // NW
K = 80
STEPS = E_TILE // K
NPAD = 10240
SLAB = NPAD // NS
ZROWS = 128

_mesh = plsc.VectorSubcoreMesh(core_axis_name="c", subcore_axis_name="s")

_sc_params = pltpu.CompilerParams()
if "needs_layout_passes" in pltpu.CompilerParams.__dataclass_fields__:
    _sc_params = dataclasses.replace(_sc_params, needs_layout_passes=False)


@functools.partial(
    pl.kernel,
    out_type=jax.ShapeDtypeStruct((NC, NPAD, L), jnp.float32),
    mesh=_mesh,
    scratch_types=[
        pltpu.VMEM((K,), jnp.int32),
        pltpu.VMEM((K,), jnp.int32),
        pltpu.VMEM((K, L), jnp.float32),
        pltpu.VMEM((K, L), jnp.float32),
        pltpu.VMEM_SHARED((NPAD, L), jnp.float32),
    ],
)
def _deg_kernel(n1_hbm, out_hbm, idx_v, riota_v, ones_v, zero_v, deg_sh):
    c = lax.axis_index("c")
    s = lax.axis_index("s")
    wid = s * NC + c
    ones16 = jnp.ones((L,), jnp.float32)
    zeros16 = jnp.zeros((L,), jnp.float32)

    @pl.loop(0, K)
    def _(i):
        ones_v[i, :] = ones16

    @pl.loop(0, K)
    def _(i):
        zero_v[i, :] = zeros16

    base = wid * E_TILE
    iota16 = lax.iota(jnp.int32, L)

    def _fill_iota(buf, start):
        @pl.loop(0, K // L)
        def _(g):
            buf[pl.ds(g * L, L)] = iota16 + (start + g * L)

    @pl.loop(0, SLAB // K)
    def _(z):
        _fill_iota(riota_v, s * SLAB + z * K)
        pltpu.sync_copy(zero_v, deg_sh.at[riota_v])

    plsc.subcore_barrier()

    @pl.loop(0, STEPS)
    def _(k):
        pltpu.sync_copy(n1_hbm.at[pl.ds(base + k * K, K)], idx_v)
        pltpu.sync_copy(ones_v, deg_sh.at[idx_v], add=True)

    plsc.subcore_barrier()

    @pl.loop(0, SLAB // K)
    def _(z):
        _fill_iota(riota_v, s * SLAB + z * K)
        pltpu.sync_copy(deg_sh.at[riota_v], zero_v)
        pltpu.sync_copy(zero_v,
                        out_hbm.at[c, pl.ds(s * SLAB + z * K, K)])


@functools.partial(
    pl.kernel,
    out_type=jax.ShapeDtypeStruct((NC, NPAD, D), jnp.float32),
    mesh=_mesh,
    scratch_types=[
        pltpu.VMEM((K,), jnp.int32),
        pltpu.VMEM((K,), jnp.int32),
        pltpu.VMEM((K,), jnp.int32),
        pltpu.VMEM((K,), jnp.float32),
        pltpu.VMEM((K,), jnp.float32),
        pltpu.VMEM((K, D), jnp.float32),
        pltpu.VMEM((K, D), jnp.float32),
        pltpu.VMEM((K, D), jnp.float32),
        pltpu.VMEM_SHARED((NPAD, D), jnp.float32),
        pltpu.SemaphoreType.DMA,
    ],
    compiler_params=_sc_params,
)
def _push_kernel(n1_hbm, n2_hbm, dist_hbm, h2_hbm, out_hbm,
                 n1_v, n2_v, riota_v, d_v, s_v, rows_v, sc_v, zero_v,
                 acc_sh, sem):
    c = lax.axis_index("c")
    s = lax.axis_index("s")
    wid = s * NC + c
    zeros16 = jnp.zeros((L,), jnp.float32)
    iota16 = lax.iota(jnp.int32, L)

    @pl.loop(0, K)
    def _(i):
        for j in range(D // L):
            zero_v[i, pl.ds(j * L, L)] = zeros16

    def _fill_iota(buf, start):
        @pl.loop(0, K // L)
        def _(g):
            buf[pl.ds(g * L, L)] = iota16 + (start + g * L)

    @pl.loop(0, SLAB // K)
    def _(z):
        _fill_iota(riota_v, s * SLAB + z * K)
        pltpu.sync_copy(zero_v, acc_sh.at[riota_v])

    plsc.subcore_barrier()

    base = wid * E_TILE

    @pl.loop(0, STEPS)
    def _(k):
        off = base + k * K
        pltpu.sync_copy(n1_hbm.at[pl.ds(off, K)], n1_v)
        pltpu.sync_copy(n2_hbm.at[pl.ds(off, K)], n2_v)
        pltpu.sync_copy(dist_hbm.at[pl.ds(off, K)], d_v)
        pltpu.async_copy(h2_hbm.at[n2_v], rows_v, sem).wait()

        @pl.loop(0, K // L)
        def _(g):
            dv = d_v[pl.ds(g * L, L)]
            s_v[pl.ds(g * L, L)] = jnp.exp(-(dv * dv))

        @pl.loop(0, K)
        def _(r):
            sw = plsc.load_gather(s_v, [jnp.full((L,), r, jnp.int32)])
            for j in range(D // L):
                sc_v[r, pl.ds(j * L, L)] = rows_v[r, pl.ds(j * L, L)] * sw

        pltpu.sync_copy(sc_v, acc_sh.at[n1_v], add=True)

    plsc.subcore_barrier()

    @pl.loop(0, SLAB // K)
    def _(z):
        _fill_iota(riota_v, s * SLAB + z * K)
        pltpu.sync_copy(acc_sh.at[riota_v], zero_v)
        pltpu.sync_copy(zero_v,
                        out_hbm.at[c, pl.ds(s * SLAB + z * K, K)])


_RB = 400


def _mm_body(poi_ref, wt_ref, b_ref, out_ref):
    h = jnp.dot(poi_ref[...], wt_ref[...],
                preferred_element_type=jnp.float32,
                precision=lax.Precision.HIGHEST)
    out_ref[...] = h + b_ref[...]


def _scale_body(hb_ref, deg_ref, out_ref):
    deg = deg_ref[0] + deg_ref[1]
    rdeg = lax.rsqrt(deg[:, 0:1])
    out_ref[...] = hb_ref[...] * rdeg


def _fin_body(p_ref, out_ref):
    t = p_ref[0] + p_ref[1]
    t = jnp.where(t >= 0, t, 0.01 * t)
    nrm = jnp.sqrt(jnp.sum(t * t, axis=1, keepdims=True))
    out_ref[...] = t / jnp.maximum(nrm, 1e-12)


def _mm_call(poi, wt, b2):
    return pl.pallas_call(
        _mm_body,
        grid=(N // _RB,),
        in_specs=[
            pl.BlockSpec((_RB, D), lambda i: (i, 0)),
            pl.BlockSpec((D, D), lambda i: (0, 0)),
            pl.BlockSpec((1, D), lambda i: (0, 0)),
        ],
        out_specs=pl.BlockSpec((_RB, D), lambda i: (i, 0)),
        out_shape=jax.ShapeDtypeStruct((N, D), jnp.float32),
    )(poi, wt, b2)


def _scale_call(hb, degp):
    return pl.pallas_call(
        _scale_body,
        grid=(N // _RB,),
        in_specs=[
            pl.BlockSpec((_RB, D), lambda i: (i, 0)),
            pl.BlockSpec((NC, _RB, L), lambda i: (0, i, 0)),
        ],
        out_specs=pl.BlockSpec((_RB, D), lambda i: (i, 0)),
        out_shape=jax.ShapeDtypeStruct((N, D), jnp.float32),
    )(hb, degp)


def _fin_call(partials):
    return pl.pallas_call(
        _fin_body,
        grid=(N // _RB,),
        in_specs=[pl.BlockSpec((NC, _RB, D), lambda i: (0, i, 0))],
        out_specs=pl.BlockSpec((_RB, D), lambda i: (i, 0)),
        out_shape=jax.ShapeDtypeStruct((N, D), jnp.float32),
    )(partials)


def kernel(poi_rep, edge_index, dist_vec, W, b):
    nodes1 = edge_index[0]
    nodes2 = edge_index[1]
    degp = _deg_kernel(nodes1)
    hb = _mm_call(poi_rep, W.T, b.reshape(1, D))
    h2 = _scale_call(hb, degp)
    partials = _push_kernel(nodes1, nodes2, dist_vec, h2)
    return _fin_call(partials)


# speedup vs baseline: 9.6691x; 1.3383x over previous
import dataclasses
import functools

import jax
import jax.numpy as jnp
from jax import lax
from jax.experimental import pallas as pl
from jax.experimental.pallas import tpu as pltpu
from jax.experimental.pallas import tpu_sc as plsc

N = 10000
E = 320000
D = 128

NC = 2
NS = 16
L = 16
NW = NC * NS
E_TILE = E // NW
K = 80
STEPS = E_TILE // K
NPAD = 10240
SLAB = NPAD // NS
ZROWS = 128

_mesh = plsc.VectorSubcoreMesh(core_axis_name="c", subcore_axis_name="s")

_sc_params = pltpu.CompilerParams()
if "needs_layout_passes" in pltpu.CompilerParams.__dataclass_fields__:
    _sc_params = dataclasses.replace(_sc_params, needs_layout_passes=False)


@functools.partial(
    pl.kernel,
    out_type=jax.ShapeDtypeStruct((NC, NPAD, L), jnp.float32),
    mesh=_mesh,
    scratch_types=[
        pltpu.VMEM((K,), jnp.int32),
        pltpu.VMEM((K,), jnp.int32),
        pltpu.VMEM((K, L), jnp.float32),
        pltpu.VMEM((K, L), jnp.float32),
        pltpu.VMEM_SHARED((NPAD, L), jnp.float32),
    ],
)
def _deg_kernel(n1_hbm, out_hbm, idx_v, riota_v, ones_v, zero_v, deg_sh):
    c = lax.axis_index("c")
    s = lax.axis_index("s")
    wid = s * NC + c
    ones16 = jnp.ones((L,), jnp.float32)
    zeros16 = jnp.zeros((L,), jnp.float32)

    @pl.loop(0, K)
    def _(i):
        ones_v[i, :] = ones16

    @pl.loop(0, K)
    def _(i):
        zero_v[i, :] = zeros16

    base = wid * E_TILE
    iota16 = lax.iota(jnp.int32, L)

    def _fill_iota(buf, start):
        @pl.loop(0, K // L)
        def _(g):
            buf[pl.ds(g * L, L)] = iota16 + (start + g * L)

    @pl.loop(0, SLAB // K)
    def _(z):
        _fill_iota(riota_v, s * SLAB + z * K)
        pltpu.sync_copy(zero_v, deg_sh.at[riota_v])

    plsc.subcore_barrier()

    @pl.loop(0, STEPS)
    def _(k):
        pltpu.sync_copy(n1_hbm.at[pl.ds(base + k * K, K)], idx_v)
        pltpu.sync_copy(ones_v, deg_sh.at[idx_v], add=True)

    plsc.subcore_barrier()

    @pl.loop(0, SLAB // K)
    def _(z):
        _fill_iota(riota_v, s * SLAB + z * K)
        pltpu.sync_copy(deg_sh.at[riota_v], zero_v)
        pltpu.sync_copy(zero_v,
                        out_hbm.at[c, pl.ds(s * SLAB + z * K, K)])


ECH = E // K


@functools.partial(
    pl.kernel,
    out_type=jax.ShapeDtypeStruct((NC, NPAD, D), jnp.float32),
    mesh=_mesh,
    scratch_types=[
        pltpu.VMEM((2, K), jnp.int32),
        pltpu.VMEM((2, K), jnp.int32),
        pltpu.VMEM((K,), jnp.float32),
        pltpu.VMEM((K,), jnp.float32),
        pltpu.VMEM((K,), jnp.int32),
        pltpu.VMEM((K,), jnp.float32),
        pltpu.VMEM((K, D), jnp.float32),
        pltpu.VMEM((K, D), jnp.float32),
        pltpu.VMEM((K, D), jnp.float32),
        pltpu.VMEM((K, D), jnp.float32),
        pltpu.VMEM_SHARED((NPAD, D), jnp.float32),
        pltpu.SemaphoreType.DMA,
        pltpu.SemaphoreType.DMA,
        pltpu.SemaphoreType.DMA,
        pltpu.SemaphoreType.DMA,
    ],
    compiler_params=_sc_params,
)
def _push_kernel(edges_hbm, dist_hbm, h2_hbm, out_hbm,
                 eidx0, eidx1, dbuf0, dbuf1, riota_v, s_v,
                 rows0, rows1, sc_v, zero_v, acc_sh,
                 semi0, semi1, semg0, semg1):
    c = lax.axis_index("c")
    s = lax.axis_index("s")
    wid = s * NC + c
    zeros16 = jnp.zeros((L,), jnp.float32)
    iota16 = lax.iota(jnp.int32, L)

    @pl.loop(0, K)
    def _(i):
        for j in range(D // L):
            zero_v[i, pl.ds(j * L, L)] = zeros16

    def _fill_iota(buf, start):
        @pl.loop(0, K // L)
        def _(g):
            buf[pl.ds(g * L, L)] = iota16 + (start + g * L)

    @pl.loop(0, SLAB // K)
    def _(z):
        _fill_iota(riota_v, s * SLAB + z * K)
        pltpu.sync_copy(zero_v, acc_sh.at[riota_v])

    plsc.subcore_barrier()

    base_ch = wid * STEPS

    def issue_idx(cid, eidx, dbuf, sem):
        pltpu.async_copy(edges_hbm.at[cid], eidx, sem)
        pltpu.async_copy(dist_hbm.at[cid], dbuf, sem)

    def wait_idx(eidx, dbuf, sem):
        pltpu.make_async_copy(edges_hbm.at[0], eidx, sem).wait()
        pltpu.make_async_copy(dist_hbm.at[0], dbuf, sem).wait()

    def issue_gather(eidx, rows, sem):
        pltpu.async_copy(h2_hbm.at[eidx.at[1]], rows, sem)

    def wait_gather(eidx, rows, sem):
        pltpu.make_async_copy(h2_hbm.at[eidx.at[1]], rows, sem).wait()

    def scale_scatter(eidx, dbuf, rows):
        @pl.loop(0, K // L)
        def _(g):
            dv = dbuf[pl.ds(g * L, L)]
            s_v[pl.ds(g * L, L)] = jnp.exp(-(dv * dv))

        @pl.loop(0, K)
        def _(r):
            sw = plsc.load_gather(s_v, [jnp.full((L,), r, jnp.int32)])
            for j in range(D // L):
                sc_v[r, pl.ds(j * L, L)] = rows[r, pl.ds(j * L, L)] * sw

        pltpu.sync_copy(sc_v, acc_sh.at[eidx.at[0]], add=True)

    issue_idx(base_ch, eidx0, dbuf0, semi0)
    wait_idx(eidx0, dbuf0, semi0)
    issue_idx(base_ch + 1, eidx1, dbuf1, semi1)
    issue_gather(eidx0, rows0, semg0)

    @pl.loop(0, (STEPS - 1) // 2)
    def _(t):
        j = 2 * t
        wait_idx(eidx1, dbuf1, semi1)
        issue_gather(eidx1, rows1, semg1)
        wait_gather(eidx0, rows0, semg0)
        scale_scatter(eidx0, dbuf0, rows0)
        issue_idx(base_ch + j + 2, eidx0, dbuf0, semi0)
        wait_idx(eidx0, dbuf0, semi0)
        issue_gather(eidx0, rows0, semg0)
        wait_gather(eidx1, rows1, semg1)
        scale_scatter(eidx1, dbuf1, rows1)
        issue_idx(base_ch + j + 3, eidx1, dbuf1, semi1)

    wait_gather(eidx0, rows0, semg0)
    scale_scatter(eidx0, dbuf0, rows0)
    wait_idx(eidx1, dbuf1, semi1)

    plsc.subcore_barrier()

    @pl.loop(0, SLAB // K)
    def _(z):
        _fill_iota(riota_v, s * SLAB + z * K)
        pltpu.sync_copy(acc_sh.at[riota_v], zero_v)
        pltpu.sync_copy(zero_v,
                        out_hbm.at[c, pl.ds(s * SLAB + z * K, K)])


_RB = 400


def _mm_body(poi_ref, wt_ref, b_ref, out_ref):
    h = jnp.dot(poi_ref[...], wt_ref[...],
                preferred_element_type=jnp.float32,
                precision=lax.Precision.HIGHEST)
    out_ref[...] = h + b_ref[...]


def _scale_body(hb_ref, deg_ref, out_ref):
    deg = deg_ref[0] + deg_ref[1]
    rdeg = lax.rsqrt(deg[:, 0:1])
    out_ref[...] = hb_ref[...] * rdeg


def _fin_body(p_ref, out_ref):
    t = p_ref[0] + p_ref[1]
    t = jnp.where(t >= 0, t, 0.01 * t)
    nrm = jnp.sqrt(jnp.sum(t * t, axis=1, keepdims=True))
    out_ref[...] = t / jnp.maximum(nrm, 1e-12)


def _mm_call(poi, wt, b2):
    return pl.pallas_call(
        _mm_body,
        grid=(N // _RB,),
        in_specs=[
            pl.BlockSpec((_RB, D), lambda i: (i, 0)),
            pl.BlockSpec((D, D), lambda i: (0, 0)),
            pl.BlockSpec((1, D), lambda i: (0, 0)),
        ],
        out_specs=pl.BlockSpec((_RB, D), lambda i: (i, 0)),
        out_shape=jax.ShapeDtypeStruct((N, D), jnp.float32),
    )(poi, wt, b2)


def _scale_call(hb, degp):
    return pl.pallas_call(
        _scale_body,
        grid=(N // _RB,),
        in_specs=[
            pl.BlockSpec((_RB, D), lambda i: (i, 0)),
            pl.BlockSpec((NC, _RB, L), lambda i: (0, i, 0)),
        ],
        out_specs=pl.BlockSpec((_RB, D), lambda i: (i, 0)),
        out_shape=jax.ShapeDtypeStruct((N, D), jnp.float32),
    )(hb, degp)


def _fin_call(partials):
    return pl.pallas_call(
        _fin_body,
        grid=(N // _RB,),
        in_specs=[pl.BlockSpec((NC, _RB, D), lambda i: (0, i, 0))],
        out_specs=pl.BlockSpec((_RB, D), lambda i: (i, 0)),
        out_shape=jax.ShapeDtypeStruct((N, D), jnp.float32),
    )(partials)


def kernel(poi_rep, edge_index, dist_vec, W, b):
    nodes1 = edge_index[0]
    nodes2 = edge_index[1]
    degp = _deg_kernel(nodes1)
    hb = _mm_call(poi_rep, W.T, b.reshape(1, D))
    h2 = _scale_call(hb, degp)
    edges_c = edge_index.reshape(2, ECH, K).transpose(1, 0, 2)
    edges_pad = jnp.concatenate(
        [edges_c, jnp.zeros((2, 2, K), jnp.int32)], axis=0)
    dist_pad = jnp.concatenate(
        [dist_vec.reshape(ECH, K), jnp.zeros((2, K), jnp.float32)], axis=0)
    partials = _push_kernel(edges_pad, dist_pad, h2)
    return _fin_call(partials)
